# baseline (speedup 1.0000x reference)
import jax
import jax.numpy as jnp
from jax import lax
from jax.experimental import pallas as pl
from jax.experimental.pallas import tpu as pltpu
from jax.experimental.pallas import tpu_sc as plsc

PROMPT_NUM = 50
PROMPT_PAD = 56
HIDDEN_SIZE = 1024
BATCH = 1024
MASK_PAD = 128

_K = 16
_NBULK = BATCH // _K


def _tc_body(table_v, emb_hbm, mask_hbm, staged, ones_v, sem, mask_sem):
    staged[...] = jnp.broadcast_to(
        table_v[...][None], (_K, PROMPT_PAD, HIDDEN_SIZE)
    )
    ones_v[...] = jnp.ones((BATCH, MASK_PAD), jnp.float32)
    bulk = [
        pltpu.make_async_copy(
            staged,
            emb_hbm.at[pl.ds(j * _K, _K), pl.ds(0, PROMPT_PAD)],
            sem,
        )
        for j in range(_NBULK)
    ]
    mask_h = pltpu.make_async_copy(
        ones_v, mask_hbm.at[:, pl.ds(0, MASK_PAD)], mask_sem
    )
    mask_h.start()
    for h in bulk:
        h.start()
    for h in bulk:
        h.wait()
    mask_h.wait()


def _tc_broadcast(prompt_table):
    tab = jnp.pad(prompt_table, ((0, PROMPT_PAD - PROMPT_NUM), (0, 0)))
    return pl.pallas_call(
        _tc_body,
        out_shape=(
            jax.ShapeDtypeStruct((BATCH, PROMPT_NUM, HIDDEN_SIZE), jnp.float32),
            jax.ShapeDtypeStruct((BATCH, PROMPT_NUM), jnp.float32),
        ),
        in_specs=[pl.BlockSpec(memory_space=pltpu.VMEM)],
        out_specs=(
            pl.BlockSpec(memory_space=pl.ANY),
            pl.BlockSpec(memory_space=pl.ANY),
        ),
        scratch_shapes=[
            pltpu.VMEM((_K, PROMPT_PAD, HIDDEN_SIZE), jnp.float32),
            pltpu.VMEM((BATCH, MASK_PAD), jnp.float32),
            pltpu.SemaphoreType.DMA,
            pltpu.SemaphoreType.DMA,
        ],
    )(tab)


def kernel(batch_size, prompt_table):
    emb, mask = _tc_broadcast(prompt_table)
    return emb, mask
